# baseline (speedup 1.0000x reference)
import functools

import jax
import jax.numpy as jnp
from jax import lax
from jax.experimental import pallas as pl
from jax.experimental.pallas import tpu as pltpu
from jax.experimental.pallas import tpu_sc as plsc

_CA = 200
_L = 16


@functools.lru_cache(maxsize=None)
def _build_relayout(V, F, num_cores, num_subcores):
    NW = num_cores * num_subcores
    M = V // _CA
    CMAX = 2 * ((M + 2 * NW - 1) // (2 * NW))
    assert M % 2 == 0 and F % _L == 0 and _CA % 8 == 0

    mesh = plsc.VectorSubcoreMesh(core_axis_name="c", subcore_axis_name="s")

    @functools.partial(
        pl.kernel,
        mesh=mesh,
        compiler_params=pltpu.CompilerParams(
            use_tc_tiling_on_sc=False, needs_layout_passes=False),
        out_type=jax.ShapeDtypeStruct((V, F), jnp.float32),
        scratch_types=[
            pltpu.VMEM((2, F, _CA), jnp.float32),
            pltpu.VMEM((2, _CA, F), jnp.float32),
            pltpu.SemaphoreType.DMA,
            pltpu.SemaphoreType.DMA,
            pltpu.SemaphoreType.DMA,
            pltpu.SemaphoreType.DMA,
        ],
    )
    def body(src_hbm, out_hbm, in_v, tr_v, si0, si1, so0, so1):
        wid = lax.axis_index("s") * num_cores + lax.axis_index("c")
        sem_i = (si0, si1)
        sem_o = (so0, so1)

        def chunk(c):
            return c * NW + wid

        def valid(c):
            return chunk(c) < M

        def in_copy(c, b):
            return pltpu.make_async_copy(
                src_hbm.at[:, chunk(c)], in_v.at[b], sem_i[b])

        def out_copy(c, b):
            return pltpu.make_async_copy(
                tr_v.at[b], out_hbm.at[pl.ds(chunk(c) * _CA, _CA)], sem_o[b])

        def transpose(b):
            def row(j, carry):
                for k in range(F // _L):
                    idx_f = jax.lax.iota(jnp.int32, _L) + (k * _L)
                    idx_j = jnp.zeros((_L,), jnp.int32) + j
                    v = plsc.load_gather(in_v.at[b], [idx_f, idx_j])
                    tr_v[b, j, pl.ds(k * _L, _L)] = v
                return carry
            lax.fori_loop(0, _CA, row, 0)

        @pl.when(valid(0))
        def _():
            in_copy(0, 0).start()

        @pl.when(valid(1))
        def _():
            in_copy(1, 1).start()

        def main(p, carry):
            for off in range(2):
                c = 2 * p + off
                b = off

                @pl.when(valid(c))
                def _():
                    in_copy(c, b).wait()

                @pl.when(jnp.logical_and(c >= 2, valid(c - 2)))
                def _():
                    out_copy(c - 2, b).wait()

                @pl.when(valid(c))
                def _():
                    transpose(b)

                @pl.when(valid(c + 2))
                def _():
                    in_copy(c + 2, b).start()

                @pl.when(valid(c))
                def _():
                    out_copy(c, b).start()
            return carry

        lax.fori_loop(0, CMAX // 2, main, 0)

        @pl.when(valid(CMAX - 2))
        def _():
            out_copy(CMAX - 2, 0).wait()

        @pl.when(valid(CMAX - 1))
        def _():
            out_copy(CMAX - 1, 1).wait()

    return body


@functools.lru_cache(maxsize=None)
def _build_gather(B, H, V, F, num_cores, num_subcores):
    NW = num_cores * num_subcores
    BPW = B // NW
    assert B % NW == 0 and H % 2 == 0 and F % _L == 0 and BPW % _L == 0

    mesh = plsc.VectorSubcoreMesh(core_axis_name="c", subcore_axis_name="s")

    @functools.partial(
        pl.kernel,
        mesh=mesh,
        compiler_params=pltpu.CompilerParams(
            use_tc_tiling_on_sc=False, needs_layout_passes=False),
        out_type=jax.ShapeDtypeStruct((H, F, B), jnp.float32),
        scratch_types=[
            pltpu.VMEM((H, BPW), jnp.int32),
            pltpu.VMEM((2, BPW, F), jnp.float32),
            pltpu.VMEM((2, F, BPW), jnp.float32),
            pltpu.SemaphoreType.DMA,
            pltpu.SemaphoreType.DMA,
            pltpu.SemaphoreType.DMA,
            pltpu.SemaphoreType.DMA,
        ],
    )
    def body(idx_hbm, table_hbm, out_hbm, idx_v, rows_v, tr_v,
             sg0, sg1, so0, so1):
        wid = lax.axis_index("s") * num_cores + lax.axis_index("c")
        sem_g = (sg0, sg1)
        sem_o = (so0, so1)
        b0 = wid * BPW
        pltpu.sync_copy(idx_hbm.at[:, pl.ds(b0, BPW)], idx_v)

        def gather(h, b):
            return pltpu.make_async_copy(
                table_hbm.at[idx_v.at[h]], rows_v.at[b], sem_g[b])

        def out_copy(h, b):
            return pltpu.make_async_copy(
                tr_v.at[b], out_hbm.at[h, :, pl.ds(b0, BPW)], sem_o[b])

        def transpose(b):
            def row(f, carry):
                for k in range(BPW // _L):
                    idx_b = jax.lax.iota(jnp.int32, _L) + (k * _L)
                    idx_f = jnp.zeros((_L,), jnp.int32) + f
                    v = plsc.load_gather(rows_v.at[b], [idx_b, idx_f])
                    tr_v[b, f, pl.ds(k * _L, _L)] = v
                return carry
            lax.fori_loop(0, F, row, 0)

        gather(0, 0).start()
        gather(1, 1).start()

        def main(p, carry):
            for off in range(2):
                h = 2 * p + off
                b = off
                gather(h, b).wait()

                @pl.when(h >= 2)
                def _():
                    out_copy(h - 2, b).wait()

                transpose(b)

                @pl.when(h + 2 < H)
                def _():
                    gather(h + 2, b).start()

                out_copy(h, b).start()
            return carry

        lax.fori_loop(0, H // 2, main, 0)
        out_copy(H - 2, 0).wait()
        out_copy(H - 1, 1).wait()

    return body


def kernel(inputs, embedding):
    B, H = inputs.shape
    V, F = embedding.shape
    info = plsc.get_sparse_core_info()
    nc, ns = info.num_cores, info.num_subcores
    emb_t = embedding.T.reshape(F, V // _CA, _CA)
    idx_t = inputs.astype(jnp.int32).T
    table_lin = _build_relayout(V, F, nc, ns)(emb_t)
    out_t = _build_gather(B, H, V, F, nc, ns)(idx_t, table_lin)
    return out_t.transpose(2, 0, 1)


# speedup vs baseline: 5.7341x; 5.7341x over previous
import functools

import jax
import jax.numpy as jnp
from jax import lax
from jax.experimental import pallas as pl
from jax.experimental.pallas import tpu as pltpu
from jax.experimental.pallas import tpu_sc as plsc

_GRP = 512
_CHUNK = 512
_NBUF = 2


@functools.lru_cache(maxsize=None)
def _build(N, F, num_cores, num_subcores):
    NW = num_cores * num_subcores
    PER_W = N // NW
    K = _CHUNK // _GRP
    NCHUNK = PER_W // _CHUNK
    IDX_ROWS = PER_W // _GRP
    assert NCHUNK >= 2 and NCHUNK % 2 == 0

    mesh = plsc.VectorSubcoreMesh(core_axis_name="c", subcore_axis_name="s")

    @functools.partial(
        pl.kernel,
        mesh=mesh,
        compiler_params=pltpu.CompilerParams(use_tc_tiling_on_sc=False),
        out_type=jax.ShapeDtypeStruct((N, F), jnp.float32),
        scratch_types=[
            pltpu.VMEM((IDX_ROWS, _GRP), jnp.int32),
            pltpu.VMEM((_NBUF * _CHUNK, F), jnp.float32),
            pltpu.SemaphoreType.DMA,
            pltpu.SemaphoreType.DMA,
            pltpu.SemaphoreType.DMA,
            pltpu.SemaphoreType.DMA,
        ],
    )
    def body(idx_hbm, table_hbm, out_hbm, idx_v, rows_v,
             sem_g0, sem_g1, sem_o0, sem_o1):
        wid = lax.axis_index("s") * num_cores + lax.axis_index("c")
        sem_g = (sem_g0, sem_g1)
        sem_o = (sem_o0, sem_o1)
        pltpu.sync_copy(idx_hbm.at[pl.ds(wid * IDX_ROWS, IDX_ROWS)], idx_v)

        def gathers(g, b):
            return [
                pltpu.make_async_copy(
                    table_hbm.at[idx_v.at[g * K + j]],
                    rows_v.at[pl.ds(b * _CHUNK + j * _GRP, _GRP)],
                    sem_g[b],
                )
                for j in range(K)
            ]

        def out_copy(g, b):
            return pltpu.make_async_copy(
                rows_v.at[pl.ds(b * _CHUNK, _CHUNK)],
                out_hbm.at[pl.ds(wid * PER_W + g * _CHUNK, _CHUNK)],
                sem_o[b],
            )

        for d in gathers(0, 0):
            d.start()
        for d in gathers(1, 1):
            d.start()
        for d in gathers(0, 0):
            d.wait()
        out_copy(0, 0).start()

        def main(go, carry):
            for off in range(2):
                g = 2 * go + 1 + off
                b = 1 - off
                out_copy(g - 1, 1 - b).wait()
                for d in gathers(g + 1, 1 - b):
                    d.start()
                for d in gathers(g, b):
                    d.wait()
                out_copy(g, b).start()
            return carry

        lax.fori_loop(0, (NCHUNK - 2) // 2, main, 0)

        out_copy(NCHUNK - 2, 0).wait()
        for d in gathers(NCHUNK - 1, 1):
            d.wait()
        out_copy(NCHUNK - 1, 1).start()
        out_copy(NCHUNK - 1, 1).wait()

    return body


def kernel(inputs, embedding):
    B, H = inputs.shape
    V, F = embedding.shape
    N = B * H
    info = plsc.get_sparse_core_info()
    idx = inputs.reshape(N // _GRP, _GRP).astype(jnp.int32)
    out = _build(N, F, info.num_cores, info.num_subcores)(idx, embedding)
    return out.reshape(B, H, F)
